# Initial kernel scaffold; baseline (speedup 1.0000x reference)
import jax
import jax.numpy as jnp
from jax.experimental import pallas as pl


def kernel(sentence, table):
    raise NotImplementedError("write your pallas kernel here")


# speedup vs baseline: 4.6376x; 4.6376x over previous
import functools

import jax
import jax.numpy as jnp
from jax import lax
from jax.experimental import pallas as pl
from jax.experimental.pallas import tpu as pltpu
from jax.experimental.pallas import tpu_sc as plsc

EMB_DIM = 32
NUM_CORES = 2
NUM_SUBCORES = 16
NUM_WORKERS = NUM_CORES * NUM_SUBCORES


@functools.lru_cache(maxsize=None)
def _make_gather(n_rows: int, chunk: int):
    rows_per_worker = n_rows // NUM_WORKERS
    n_chunks = rows_per_worker // chunk
    mesh = plsc.VectorSubcoreMesh(core_axis_name="c", subcore_axis_name="s")

    @functools.partial(
        pl.kernel,
        mesh=mesh,
        compiler_params=pltpu.CompilerParams(use_tc_tiling_on_sc=False),
        out_type=jax.ShapeDtypeStruct((n_rows, EMB_DIM), jnp.float32),
        scratch_types=[
            pltpu.VMEM((chunk,), jnp.int32),
            pltpu.VMEM((chunk, EMB_DIM), jnp.float32),
            pltpu.SemaphoreType.DMA,
        ],
    )
    def gather_kernel(idx_hbm, table_hbm, out_hbm, idx_v, rows_v, sem):
        wid = lax.axis_index("s") * NUM_CORES + lax.axis_index("c")
        wbase = wid * rows_per_worker
        for i in range(n_chunks):
            base = wbase + i * chunk
            pltpu.sync_copy(idx_hbm.at[pl.ds(base, chunk)], idx_v)
            pltpu.async_copy(table_hbm.at[idx_v], rows_v, sem).wait()
            pltpu.sync_copy(rows_v, out_hbm.at[pl.ds(base, chunk)])

    return gather_kernel


def kernel(sentence, table):
    batch, seq = sentence.shape
    idx = sentence.reshape(-1).astype(jnp.int32)
    out = _make_gather(idx.shape[0], 1600)(idx, table)
    return out.reshape(batch, seq, EMB_DIM)


# speedup vs baseline: 4.6464x; 1.0019x over previous
import functools

import jax
import jax.numpy as jnp
from jax import lax
from jax.experimental import pallas as pl
from jax.experimental.pallas import tpu as pltpu
from jax.experimental.pallas import tpu_sc as plsc

EMB_DIM = 32
NUM_CORES = 2
NUM_SUBCORES = 16
NUM_WORKERS = NUM_CORES * NUM_SUBCORES


@functools.lru_cache(maxsize=None)
def _make_gather(n_rows: int, chunk: int):
    rows_per_worker = n_rows // NUM_WORKERS
    n_chunks = rows_per_worker // chunk
    mesh = plsc.VectorSubcoreMesh(core_axis_name="c", subcore_axis_name="s")

    @functools.partial(
        pl.kernel,
        mesh=mesh,
        compiler_params=pltpu.CompilerParams(use_tc_tiling_on_sc=False),
        out_type=jax.ShapeDtypeStruct((n_rows, EMB_DIM), jnp.float32),
        scratch_types=[
            pltpu.VMEM((n_chunks, chunk), jnp.int32),
            pltpu.VMEM((chunk, EMB_DIM), jnp.float32),
            pltpu.VMEM((chunk, EMB_DIM), jnp.float32),
            pltpu.SemaphoreType.DMA,
            pltpu.SemaphoreType.DMA,
            pltpu.SemaphoreType.DMA,
            pltpu.SemaphoreType.DMA,
        ],
    )
    def gather_kernel(idx_hbm, table_hbm, out_hbm, idx_v, rows0, rows1,
                      g0, g1, s0, s1):
        wid = lax.axis_index("s") * NUM_CORES + lax.axis_index("c")
        wbase = wid * rows_per_worker
        pltpu.sync_copy(idx_hbm.at[pl.ds(wid * n_chunks, n_chunks)], idx_v)

        bufs = (rows0, rows1)
        gsems = (g0, g1)
        ssems = (s0, s1)
        gh = [None] * n_chunks
        sh = [None] * n_chunks
        gh[0] = pltpu.async_copy(table_hbm.at[idx_v.at[0]], bufs[0], gsems[0])
        for i in range(n_chunks):
            cur = i % 2
            if i + 1 < n_chunks:
                nxt = (i + 1) % 2
                if i >= 1:
                    sh[i - 1].wait()
                gh[i + 1] = pltpu.async_copy(
                    table_hbm.at[idx_v.at[i + 1]], bufs[nxt], gsems[nxt])
            gh[i].wait()
            sh[i] = pltpu.async_copy(
                bufs[cur], out_hbm.at[pl.ds(wbase + i * chunk, chunk)],
                ssems[cur])
        if n_chunks >= 2:
            sh[n_chunks - 2].wait()
        sh[n_chunks - 1].wait()

    return gather_kernel


def kernel(sentence, table):
    batch, seq = sentence.shape
    n_rows = batch * seq
    chunk = 1280
    idx = sentence.reshape(n_rows // chunk, chunk).astype(jnp.int32)
    out = _make_gather(n_rows, chunk)(idx, table)
    return out.reshape(batch, seq, EMB_DIM)
